# baseline (speedup 1.0000x reference)
import functools

import jax
import jax.numpy as jnp
from jax import lax
from jax.experimental import pallas as pl
from jax.experimental.pallas import tpu as pltpu
from jax.experimental.pallas import tpu_sc as plsc

_VOCAB = 100000
_EMBED = 128
_BATCH = 4096
_SEQ = 200
_SEQP = 256

_NC = 2
_NS = 16
_NW = _NC * _NS
_ROWS_PER_W = _BATCH // _NW

_CHUNK_OFFS = tuple(range(0, 192, 16)) + (184,)


def _fused(x2, W):
    mesh = plsc.VectorSubcoreMesh(core_axis_name="c", subcore_axis_name="s")

    @functools.partial(
        pl.kernel,
        mesh=mesh,
        compiler_params=pltpu.CompilerParams(needs_layout_passes=False),
        out_type=jax.ShapeDtypeStruct((_BATCH, _EMBED, _SEQ), jnp.float32),
        scratch_types=[
            pltpu.VMEM((2, 128), jnp.int32),
            pltpu.VMEM((_SEQP, _EMBED), jnp.float32),
            pltpu.VMEM((_EMBED, _SEQ), jnp.float32),
            pltpu.SemaphoreType.DMA,
        ],
    )
    def k(x_hbm, w_hbm, out_hbm, xrow_v, emb_v, obuf_v, sem):
        wid = lax.axis_index("s") * _NC + lax.axis_index("c")
        b0 = wid * _ROWS_PER_W
        iota = lax.iota(jnp.int32, 16)

        def row_body(b, carry):
            brow = b0 + b
            pltpu.sync_copy(x_hbm.at[brow], xrow_v)
            g0 = pltpu.async_copy(
                w_hbm.at[xrow_v.at[0]], emb_v.at[pl.ds(0, 128)], sem
            )
            g1 = pltpu.async_copy(
                w_hbm.at[xrow_v.at[1]], emb_v.at[pl.ds(128, 128)], sem
            )
            g0.wait()
            g1.wait()

            def l_body(l, carry2):
                lvec = jnp.full((16,), l, jnp.int32)
                for k in range(_EMBED // 16):
                    dvec = iota + (16 * k)
                    v = emb_v[l, pl.ds(16 * k, 16)]
                    plsc.store_scatter(obuf_v, [dvec, lvec], v)
                return carry2

            lax.fori_loop(0, _SEQ, l_body, 0)
            pltpu.sync_copy(obuf_v, out_hbm.at[brow])
            return carry

        lax.fori_loop(0, _ROWS_PER_W, row_body, 0)

    return k(x2, W)


def kernel(x, W):
    x2 = jnp.pad(x, ((0, 0), (0, _SEQP - _SEQ))).reshape(_BATCH, 2, 128)
    return _fused(x2, W)
